# baseline (speedup 1.0000x reference)
import functools

import jax
import jax.numpy as jnp
from jax import lax
from jax.experimental import pallas as pl
from jax.experimental.pallas import tpu as pltpu
from jax.experimental.pallas import tpu_sc as plsc

PAD = 1
BATCH = 4
SEQ = 8192
DIM = 1024
TOK = BATCH * SEQ
NCORE = 2
NSUB = 16
PW = TOK // (NCORE * NSUB)
L = 16
K = 16
NCHUNK = PW // K
NBUF = 7

_mesh = plsc.VectorSubcoreMesh(core_axis_name="c", subcore_axis_name="s")


@functools.partial(
    pl.kernel,
    out_type=jax.ShapeDtypeStruct((TOK, DIM), jnp.float32),
    mesh=_mesh,
    scratch_types=[
        pltpu.VMEM((SEQ,), jnp.int32),
        pltpu.VMEM((NCHUNK, K), jnp.int32),
        pltpu.VMEM((NBUF, K, DIM), jnp.float32),
        pltpu.SemaphoreType.DMA((NBUF,)),
        pltpu.SemaphoreType.DMA((NBUF,)),
    ],
    compiler_params=pltpu.CompilerParams(needs_layout_passes=False),
)
def _emb_lookup(ids_hbm, w_hbm, out_hbm, ids_v, pos_v, bufs, gsem, ssem):
    c = lax.axis_index("c")
    s = lax.axis_index("s")
    r = 2 * c + s // 8
    j = s % 8
    row0 = r * SEQ
    tok0 = row0 + j * PW

    pltpu.sync_copy(ids_hbm.at[pl.ds(row0, SEQ)], ids_v)

    ones = jnp.full((L,), 1, jnp.int32)
    zeros = jnp.full((L,), 0, jnp.int32)
    pads = jnp.full((L,), PAD, jnp.int32)

    def mask_at(off):
        return jnp.where(ids_v[pl.ds(off, L)] != pads, ones, zeros)

    def count_body(i, tot):
        return tot + jnp.sum(mask_at(i * L))

    carry = lax.fori_loop(0, j * (PW // L), count_body, jnp.int32(0))

    def pos_body(cidx, tot):
        for h in range(K // L):
            m = mask_at(j * PW + cidx * K + h * L)
            cs = plsc.cumsum(m)
            tot_v = jnp.full((L,), tot, jnp.int32)
            pos_v[cidx, pl.ds(h * L, L)] = (cs + tot_v) * m + ones
            tot = tot + jnp.sum(m)
        return tot

    lax.fori_loop(0, NCHUNK, pos_body, carry)

    def g_desc(ch, b):
        idx = pos_v.at[ch]
        return pltpu.make_async_copy(w_hbm.at[idx], bufs.at[b], gsem.at[b])

    def s_desc(ch, b):
        return pltpu.make_async_copy(
            bufs.at[b], out_hbm.at[pl.ds(tok0 + ch * K, K)], ssem.at[b]
        )

    for b in range(NBUF):
        g_desc(b, b).start()

    def chunk_body(i, _):
        b = i % NBUF
        g_desc(i, b).wait()
        s_desc(i, b).start()
        nxt = i + 1

        @pl.when(jnp.logical_and(nxt >= NBUF, nxt < NCHUNK))
        def _():
            nb = nxt % NBUF
            s_desc(nxt - NBUF, nb).wait()
            g_desc(nxt, nb).start()

        return 0

    lax.fori_loop(0, NCHUNK, chunk_body, 0)

    for b in range(NBUF):
        ch = NCHUNK - NBUF + b
        s_desc(ch, ch % NBUF).wait()


def kernel(input, weights):
    bsz, seq_len, _ = input.shape
    ids = input.reshape(-1).astype(jnp.int32)
    out = _emb_lookup(ids, weights)
    return out.reshape(bsz, seq_len, DIM)


# speedup vs baseline: 1.1388x; 1.1388x over previous
import functools

import jax
import jax.numpy as jnp
from jax import lax
from jax.experimental import pallas as pl
from jax.experimental.pallas import tpu as pltpu
from jax.experimental.pallas import tpu_sc as plsc

PAD = 1
BATCH = 4
SEQ = 8192
DIM = 1024
TOK = BATCH * SEQ
NCORE = 2
NSUB = 16
PW = TOK // (NCORE * NSUB)
L = 16
K = 16
NCHUNK = PW // K
NBUF = 4

_mesh = plsc.VectorSubcoreMesh(core_axis_name="c", subcore_axis_name="s")


@functools.partial(
    pl.kernel,
    out_type=jax.ShapeDtypeStruct((TOK, DIM), jnp.float32),
    mesh=_mesh,
    scratch_types=[
        pltpu.VMEM((SEQ,), jnp.int32),
        pltpu.VMEM((NCHUNK, K), jnp.int32),
        pltpu.VMEM((NBUF, K, DIM), jnp.float32),
        pltpu.SemaphoreType.DMA((NBUF,)),
        pltpu.SemaphoreType.DMA((NBUF,)),
    ],
    compiler_params=pltpu.CompilerParams(needs_layout_passes=False),
)
def _emb_lookup(ids_hbm, w_hbm, out_hbm, ids_v, pos_v, bufs, gsem, ssem):
    c = lax.axis_index("c")
    s = lax.axis_index("s")
    r = 2 * c + s // 8
    j = s % 8
    row0 = r * SEQ
    tok0 = row0 + j * PW

    pltpu.sync_copy(ids_hbm.at[pl.ds(row0, SEQ)], ids_v)

    ones = jnp.full((L,), 1, jnp.int32)
    zeros = jnp.full((L,), 0, jnp.int32)
    pads = jnp.full((L,), PAD, jnp.int32)

    def mask_at(off):
        return jnp.where(ids_v[pl.ds(off, L)] != pads, ones, zeros)

    def count_body(i, acc):
        return acc + mask_at(i * L)

    acc = lax.fori_loop(0, j * (PW // L), count_body, zeros)
    carry = jnp.sum(acc)

    def pos_body(cidx, tot):
        for h in range(K // L):
            m = mask_at(j * PW + cidx * K + h * L)
            cs = plsc.cumsum(m)
            off_v = jnp.full((L,), tot, jnp.int32)
            pos_v[cidx, pl.ds(h * L, L)] = (cs + off_v) * m + ones
            tot = tot + jnp.sum(m)
        return tot

    lax.fori_loop(0, NCHUNK, pos_body, carry)

    def g_desc(ch, b):
        idx = pos_v.at[ch]
        return pltpu.make_async_copy(w_hbm.at[idx], bufs.at[b], gsem.at[b])

    def s_desc(ch, b):
        return pltpu.make_async_copy(
            bufs.at[b], out_hbm.at[pl.ds(tok0 + ch * K, K)], ssem.at[b]
        )

    for b in range(NBUF):
        g_desc(b, b).start()

    def chunk_group(g, _):
        base = g * NBUF
        for b in range(NBUF):
            g_desc(base + b, b).wait()
            s_desc(base + b, b).start()
        for b in range(NBUF):
            s_desc(base + b, b).wait()
            g_desc(base + NBUF + b, b).start()
        return 0

    lax.fori_loop(0, NCHUNK // NBUF - 1, chunk_group, 0)

    last = NCHUNK - NBUF
    for b in range(NBUF):
        g_desc(last + b, b).wait()
        s_desc(last + b, b).start()
    for b in range(NBUF):
        s_desc(last + b, b).wait()


def kernel(input, weights):
    bsz, seq_len, _ = input.shape
    ids = input.reshape(-1).astype(jnp.int32)
    out = _emb_lookup(ids, weights)
    return out.reshape(bsz, seq_len, DIM)


# speedup vs baseline: 1.1427x; 1.0035x over previous
import functools

import jax
import jax.numpy as jnp
from jax import lax
from jax.experimental import pallas as pl
from jax.experimental.pallas import tpu as pltpu
from jax.experimental.pallas import tpu_sc as plsc

PAD = 1
BATCH = 4
SEQ = 8192
DIM = 1024
TOK = BATCH * SEQ
NCORE = 2
NSUB = 16
PW = TOK // (NCORE * NSUB)
L = 16
K = 16
NCHUNK = PW // K
NBUF = 4

_mesh = plsc.VectorSubcoreMesh(core_axis_name="c", subcore_axis_name="s")


@functools.partial(
    pl.kernel,
    out_type=jax.ShapeDtypeStruct((TOK, DIM), jnp.float32),
    mesh=_mesh,
    scratch_types=[
        pltpu.VMEM((SEQ,), jnp.int32),
        pltpu.VMEM((NBUF, K, DIM), jnp.float32),
        pltpu.SemaphoreType.DMA((NBUF,)),
        pltpu.SemaphoreType.DMA((NBUF,)),
    ],
    compiler_params=pltpu.CompilerParams(needs_layout_passes=False),
)
def _emb_lookup(ids_hbm, w_hbm, out_hbm, ids_v, bufs, gsem, ssem):
    c = lax.axis_index("c")
    s = lax.axis_index("s")
    r = 2 * c + s // 8
    j = s % 8
    row0 = r * SEQ
    tok0 = row0 + j * PW

    pltpu.sync_copy(ids_hbm.at[pl.ds(row0, SEQ)], ids_v)

    ones = jnp.full((L,), 1, jnp.int32)
    zeros = jnp.full((L,), 0, jnp.int32)
    pads = jnp.full((L,), PAD, jnp.int32)

    def mask_at(off):
        return jnp.where(ids_v[pl.ds(off, L)] != pads, ones, zeros)

    def count_body(i, acc):
        return acc + mask_at(i * L)

    acc = lax.fori_loop(0, j * (PW // L), count_body, zeros)
    carry = jnp.sum(acc)

    def chunk_idx(ch, tot):
        m = mask_at(j * PW + ch * L)
        cs = plsc.cumsum(m)
        off_v = jnp.full((L,), tot, jnp.int32)
        return (cs + off_v) * m + ones, tot + jnp.sum(m)

    def g_start(ch, b, tot):
        idx, tot = chunk_idx(ch, tot)
        pltpu.async_copy(w_hbm.at[idx], bufs.at[b], gsem.at[b])
        return tot

    def g_wait(b):
        pltpu.make_async_copy(w_hbm.at[pl.ds(0, K)], bufs.at[b],
                              gsem.at[b]).wait()

    def s_desc(ch, b):
        return pltpu.make_async_copy(
            bufs.at[b], out_hbm.at[pl.ds(tok0 + ch * K, K)], ssem.at[b]
        )

    tot = carry
    for b in range(NBUF):
        tot = g_start(b, b, tot)

    def chunk_group(g, tot):
        base = g * NBUF
        for b in range(NBUF):
            g_wait(b)
            s_desc(base + b, b).start()
        for b in range(NBUF):
            s_desc(base + b, b).wait()
            tot = g_start(base + NBUF + b, b, tot)
        return tot

    lax.fori_loop(0, NCHUNK // NBUF - 1, chunk_group, tot)

    last = NCHUNK - NBUF
    for b in range(NBUF):
        g_wait(b)
        s_desc(last + b, b).start()
    for b in range(NBUF):
        s_desc(last + b, b).wait()


def kernel(input, weights):
    bsz, seq_len, _ = input.shape
    ids = input.reshape(-1).astype(jnp.int32)
    out = _emb_lookup(ids, weights)
    return out.reshape(bsz, seq_len, DIM)
